# baseline (speedup 1.0000x reference)
import functools

import jax
import jax.numpy as jnp
from jax import lax
from jax.experimental import pallas as pl
from jax.experimental.pallas import tpu as pltpu
from jax.experimental.pallas import tpu_sc as plsc

N = 10000
E = 320000
IN_CH = 128
HID_CH = 128
OUT_CH = 64

NC = 2
NS = 16
NW = NC * NS

N_PAD = 10240
ROWS_PER_TILE = N_PAD // NS
CHUNK = 128
EC = -(-E // CHUNK)
EC_PAD = 2560
E_PAD = EC_PAD * CHUNK
CH_PER_TILE = EC_PAD // NW
CH_PER_TILE_AGG = EC_PAD // NS
HALF = HID_CH // 2
KD = 8

SLOTS = 4
KS = 1
_G2 = CH_PER_TILE_AGG // KS
_SUPER = _G2 // SLOTS

@functools.lru_cache(maxsize=1)
def _sc_kernels():
    mesh = plsc.VectorSubcoreMesh(core_axis_name="c", subcore_axis_name="s",
                                  num_cores=NC, num_subcores=NS)
    deg_kernel = pl.kernel(
        _deg_body,
        out_type=jax.ShapeDtypeStruct((NC, N_PAD, 16), jnp.float32),
        mesh=mesh,
        scratch_types=[
            pltpu.VMEM((KD, CHUNK), jnp.int32),
            pltpu.VMEM((CHUNK, 16), jnp.float32),
            pltpu.VMEM_SHARED((N_PAD, 16), jnp.float32),
            pltpu.SemaphoreType.DMA,
        ],
        compiler_params=pltpu.CompilerParams(use_tc_tiling_on_sc=False),
    )
    agg_kernel = pl.kernel(
        _agg_body,
        out_type=jax.ShapeDtypeStruct((NC, N_PAD, HALF), jnp.float32),
        mesh=mesh,
        scratch_types=(
            [pltpu.VMEM((CH_PER_TILE_AGG, CHUNK), jnp.int32),
             pltpu.VMEM((CH_PER_TILE_AGG, CHUNK), jnp.int32),
             pltpu.VMEM((SLOTS, KS, CHUNK, HALF), jnp.float32),
             pltpu.VMEM_SHARED((N_PAD, HALF), jnp.float32)]
            + [pltpu.SemaphoreType.DMA] * (2 * SLOTS)
        ),
        compiler_params=pltpu.CompilerParams(use_tc_tiling_on_sc=False),
    )
    return deg_kernel, agg_kernel


def _zero_vmem_rows(buf, nrows, width):
    zv = jnp.zeros((16,), jnp.float32)

    def row(i, _):
        for j in range(width // 16):
            buf[i, pl.ds(j * 16, 16)] = zv
        return 0

    lax.fori_loop(0, nrows, row, 0, unroll=False)


def _deg_body(dstm_hbm, out_hbm, dst_idx, ones_buf, acc16, dsem):
    c = lax.axis_index("c")
    s = lax.axis_index("s")
    wid = c * NS + s

    _zero_vmem_rows(ones_buf, CHUNK, 16)
    for r in range(ROWS_PER_TILE // CHUNK):
        pltpu.sync_copy(ones_buf,
                        acc16.at[pl.ds(s * ROWS_PER_TILE + r * CHUNK, CHUNK)])
    plsc.subcore_barrier()

    e0 = jnp.where(lax.iota(jnp.int32, 16) == 0, 1.0, 0.0).astype(jnp.float32)

    def fill(i, _):
        ones_buf[i, :] = e0
        return 0

    lax.fori_loop(0, CHUNK, fill, 0, unroll=False)

    def group(g, _):
        row0 = wid * CH_PER_TILE + g * KD
        pltpu.sync_copy(dstm_hbm.at[pl.ds(row0, KD)], dst_idx)
        for j in range(KD):
            pltpu.async_copy(ones_buf, acc16.at[dst_idx.at[j]], dsem,
                             add=True)
        for j in range(KD):
            pltpu.make_async_copy(ones_buf, acc16.at[dst_idx.at[j]],
                                  dsem).wait()
        return 0

    lax.fori_loop(0, CH_PER_TILE // KD, group, 0, unroll=False)
    plsc.subcore_barrier()

    pltpu.sync_copy(acc16.at[pl.ds(s * ROWS_PER_TILE, ROWS_PER_TILE)],
                    out_hbm.at[c, pl.ds(s * ROWS_PER_TILE, ROWS_PER_TILE)])


def _agg_body(hs2x_hbm, srcm_hbm, dstm_hbm, out_hbm,
              src_sl, dst_sl, rows, acc, *sems):
    c = lax.axis_index("c")
    s = lax.axis_index("s")
    gsem = sems[:SLOTS]
    ssem = sems[SLOTS:]

    _zero_vmem_rows(rows.at[0, 0], CHUNK, HALF)
    for r in range(ROWS_PER_TILE // CHUNK):
        pltpu.sync_copy(rows.at[0, 0],
                        acc.at[pl.ds(s * ROWS_PER_TILE + r * CHUNK, CHUNK)])

    base = s * CH_PER_TILE_AGG
    _STEP = 8
    for b in range(CH_PER_TILE_AGG // _STEP):
        pltpu.sync_copy(srcm_hbm.at[pl.ds(base + b * _STEP, _STEP)],
                        src_sl.at[pl.ds(b * _STEP, _STEP)])
        pltpu.sync_copy(dstm_hbm.at[pl.ds(base + b * _STEP, _STEP)],
                        dst_sl.at[pl.ds(b * _STEP, _STEP)])

    def tr(i, _):
        for t in range(CHUNK // 16):
            v = src_sl[i, pl.ds(t * 16, 16)]
            src_sl[i, pl.ds(t * 16, 16)] = v * 2 + c
        return 0

    lax.fori_loop(0, CH_PER_TILE_AGG, tr, 0, unroll=False)
    plsc.subcore_barrier()

    def fire_gather(i, g):
        for j in range(KS):
            pltpu.async_copy(hs2x_hbm.at[src_sl.at[g * KS + j]],
                             rows.at[i, j], gsem[i])

    def wait_gather(i, g):
        for j in range(KS):
            pltpu.make_async_copy(hs2x_hbm.at[src_sl.at[g * KS + j]],
                                  rows.at[i, j], gsem[i]).wait()

    def fire_scatter(i, g):
        for j in range(KS):
            pltpu.async_copy(rows.at[i, j],
                             acc.at[dst_sl.at[g * KS + j]], ssem[i], add=True)

    def wait_scatter(i, g):
        for j in range(KS):
            pltpu.make_async_copy(rows.at[i, j],
                                  acc.at[dst_sl.at[g * KS + j]],
                                  ssem[i]).wait()

    fire_gather(0, 0)
    fire_gather(1, 1)

    def super_group(go, _):
        for i in range(SLOTS):
            g = SLOTS * go + i
            nxt = (i + 2) % SLOTS
            wait_gather(i, g)
            fire_scatter(i, g)
            if i < 2:
                @pl.when(go >= 1)
                def _():
                    wait_scatter(nxt, g - 2)
            else:
                wait_scatter(nxt, g - 2)

            @pl.when(g + 2 <= _G2 - 1)
            def _():
                fire_gather(nxt, g + 2)
        return 0

    lax.fori_loop(0, _SUPER, super_group, 0, unroll=False)
    wait_scatter(2, _G2 - 2)
    wait_scatter(3, _G2 - 1)
    plsc.subcore_barrier()

    pltpu.sync_copy(acc.at[pl.ds(s * ROWS_PER_TILE, ROWS_PER_TILE)],
                    out_hbm.at[c, pl.ds(s * ROWS_PER_TILE, ROWS_PER_TILE)])


_TB = 1000


def _dinv_block(degp_ref):
    deg = degp_ref[0][:, 0:1] + degp_ref[1][:, 0:1] + 1.0
    return lax.rsqrt(deg)


def _scale_mm_body(degp_ref, x_ref, w_ref, out_ref):
    dinv = _dinv_block(degp_ref)
    h = jnp.dot(x_ref[...], w_ref[...], preferred_element_type=jnp.float32)
    out_ref[...] = h * dinv


def _mid_body(degp_ref, accp_ref, hs_ref, b_ref, w_ref, out_ref):
    dinv = _dinv_block(degp_ref)
    acc = jnp.concatenate([accp_ref[0], accp_ref[1]], axis=-1)
    h = jnp.maximum(dinv * (acc + hs_ref[...]) + b_ref[...], 0.0)
    out_ref[...] = jnp.dot(h, w_ref[...],
                           preferred_element_type=jnp.float32) * dinv


def _final_body(degp_ref, accp_ref, hs_ref, b_ref, mu_ref, lv_ref):
    dinv = _dinv_block(degp_ref)
    acc = jnp.concatenate([accp_ref[0], accp_ref[1]], axis=-1)
    out = dinv * (acc + hs_ref[...]) + b_ref[...]
    mu_ref[...] = out[:, :OUT_CH]
    lv_ref[...] = out[:, OUT_CH:]


_degp_spec = pl.BlockSpec((2, _TB, 16), lambda i: (0, i, 0))
_row_spec = pl.BlockSpec((_TB, HID_CH), lambda i: (i, 0))
_accp_spec = pl.BlockSpec((2, _TB, HALF), lambda i: (0, i, 0))
_w_spec = pl.BlockSpec((HID_CH, HID_CH), lambda i: (0, 0))
_b_spec = pl.BlockSpec((1, HID_CH), lambda i: (0, 0))
_half_spec = pl.BlockSpec((_TB, OUT_CH), lambda i: (i, 0))
_grid = (N // _TB,)
_row_out = jax.ShapeDtypeStruct((N, HID_CH), jnp.float32)

_scale_mm = pl.pallas_call(
    _scale_mm_body,
    grid=_grid,
    in_specs=[_degp_spec, _row_spec, _w_spec],
    out_specs=_row_spec,
    out_shape=_row_out,
)

_mid = pl.pallas_call(
    _mid_body,
    grid=_grid,
    in_specs=[_degp_spec, _accp_spec, _row_spec, _b_spec, _w_spec],
    out_specs=_row_spec,
    out_shape=_row_out,
)

_final = pl.pallas_call(
    _final_body,
    grid=_grid,
    in_specs=[_degp_spec, _accp_spec, _row_spec, _b_spec],
    out_specs=[_half_spec, _half_spec],
    out_shape=[jax.ShapeDtypeStruct((N, OUT_CH), jnp.float32),
               jax.ShapeDtypeStruct((N, OUT_CH), jnp.float32)],
)


def kernel(x, edge_index, W_shared, b_shared, W_mu, b_mu, W_logvar, b_logvar):
    npad_e = E_PAD - E
    pad_dst = N + (jnp.arange(npad_e, dtype=jnp.int32) % (N_PAD - N))
    pad_src = jnp.arange(npad_e, dtype=jnp.int32) % N
    srcm = jnp.concatenate([edge_index[0], pad_src]).reshape(EC_PAD, CHUNK)
    dstm = jnp.concatenate([edge_index[1], pad_dst]).reshape(EC_PAD, CHUNK)

    W_cat = jnp.concatenate([W_mu, W_logvar], axis=1)
    b_cat = jnp.concatenate([b_mu, b_logvar]).reshape(1, HID_CH)
    b_sh = b_shared.reshape(1, HID_CH)

    deg_kernel, agg_kernel = _sc_kernels()

    degp = deg_kernel(dstm)
    hs1 = _scale_mm(degp, x, W_shared)
    acc1 = agg_kernel(hs1.reshape(2 * N, HALF), srcm, dstm)
    hs2 = _mid(degp, acc1, hs1, b_sh, W_cat)
    acc2 = agg_kernel(hs2.reshape(2 * N, HALF), srcm, dstm)
    mu, logvar = _final(degp, acc2, hs2, b_cat)
    return (mu, logvar)


# speedup vs baseline: 1.1141x; 1.1141x over previous
import functools

import jax
import jax.numpy as jnp
from jax import lax
from jax.experimental import pallas as pl
from jax.experimental.pallas import tpu as pltpu
from jax.experimental.pallas import tpu_sc as plsc

N = 10000
E = 320000
IN_CH = 128
HID_CH = 128
OUT_CH = 64

NC = 2
NS = 16
NW = NC * NS

N_PAD = 10240
ROWS_PER_TILE = N_PAD // NS
CHUNK = 128
EC = -(-E // CHUNK)
EC_PAD = 2560
E_PAD = EC_PAD * CHUNK
CH_PER_TILE = EC_PAD // NW
CH_PER_TILE_AGG = EC_PAD // NS
HALF = HID_CH // 2
KD = 8

SLOTS = 4
KS = 2
_G2 = CH_PER_TILE_AGG // KS
_SUPER = _G2 // SLOTS

@functools.lru_cache(maxsize=1)
def _sc_kernels():
    mesh = plsc.VectorSubcoreMesh(core_axis_name="c", subcore_axis_name="s",
                                  num_cores=NC, num_subcores=NS)
    deg_kernel = pl.kernel(
        _deg_body,
        out_type=jax.ShapeDtypeStruct((NC, N_PAD, 16), jnp.float32),
        mesh=mesh,
        scratch_types=[
            pltpu.VMEM((KD, CHUNK), jnp.int32),
            pltpu.VMEM((CHUNK, 16), jnp.float32),
            pltpu.VMEM_SHARED((N_PAD, 16), jnp.float32),
            pltpu.SemaphoreType.DMA,
        ],
        compiler_params=pltpu.CompilerParams(use_tc_tiling_on_sc=False),
    )
    agg_kernel = pl.kernel(
        _agg_body,
        out_type=jax.ShapeDtypeStruct((NC, N_PAD, HALF), jnp.float32),
        mesh=mesh,
        scratch_types=(
            [pltpu.VMEM((SLOTS, KS, CHUNK), jnp.int32),
             pltpu.VMEM((SLOTS, KS, CHUNK), jnp.int32),
             pltpu.VMEM((SLOTS, KS, CHUNK, HALF), jnp.float32),
             pltpu.VMEM_SHARED((N_PAD, HALF), jnp.float32)]
            + [pltpu.SemaphoreType.DMA] * (2 * SLOTS)
        ),
        compiler_params=pltpu.CompilerParams(use_tc_tiling_on_sc=False),
    )
    return deg_kernel, agg_kernel


def _zero_vmem_rows(buf, nrows, width):
    zv = jnp.zeros((16,), jnp.float32)

    def row(i, _):
        for j in range(width // 16):
            buf[i, pl.ds(j * 16, 16)] = zv
        return 0

    lax.fori_loop(0, nrows, row, 0, unroll=False)


def _deg_body(dstm_hbm, out_hbm, dst_idx, ones_buf, acc16, dsem):
    c = lax.axis_index("c")
    s = lax.axis_index("s")
    wid = c * NS + s

    _zero_vmem_rows(ones_buf, CHUNK, 16)
    for r in range(ROWS_PER_TILE // CHUNK):
        pltpu.sync_copy(ones_buf,
                        acc16.at[pl.ds(s * ROWS_PER_TILE + r * CHUNK, CHUNK)])
    plsc.subcore_barrier()

    e0 = jnp.where(lax.iota(jnp.int32, 16) == 0, 1.0, 0.0).astype(jnp.float32)

    def fill(i, _):
        ones_buf[i, :] = e0
        return 0

    lax.fori_loop(0, CHUNK, fill, 0, unroll=False)

    def group(g, _):
        row0 = wid * CH_PER_TILE + g * KD
        pltpu.sync_copy(dstm_hbm.at[pl.ds(row0, KD)], dst_idx)
        for j in range(KD):
            pltpu.async_copy(ones_buf, acc16.at[dst_idx.at[j]], dsem,
                             add=True)
        for j in range(KD):
            pltpu.make_async_copy(ones_buf, acc16.at[dst_idx.at[j]],
                                  dsem).wait()
        return 0

    lax.fori_loop(0, CH_PER_TILE // KD, group, 0, unroll=False)
    plsc.subcore_barrier()

    pltpu.sync_copy(acc16.at[pl.ds(s * ROWS_PER_TILE, ROWS_PER_TILE)],
                    out_hbm.at[c, pl.ds(s * ROWS_PER_TILE, ROWS_PER_TILE)])


def _agg_body(hs2x_hbm, srcm_hbm, dstm_hbm, out_hbm,
              src_idx, dst_idx, rows, acc, *sems):
    c = lax.axis_index("c")
    s = lax.axis_index("s")
    gsem = sems[:SLOTS]
    ssem = sems[SLOTS:]

    _zero_vmem_rows(rows.at[0, 0], CHUNK, HALF)
    for r in range(ROWS_PER_TILE // CHUNK):
        pltpu.sync_copy(rows.at[0, 0],
                        acc.at[pl.ds(s * ROWS_PER_TILE + r * CHUNK, CHUNK)])
    plsc.subcore_barrier()

    base = s * CH_PER_TILE_AGG

    def load_idx(i, g):
        pltpu.sync_copy(srcm_hbm.at[pl.ds(base + g * KS, KS)], src_idx.at[i])
        pltpu.sync_copy(dstm_hbm.at[pl.ds(base + g * KS, KS)], dst_idx.at[i])
        for j in range(KS):
            for t in range(CHUNK // 16):
                v = src_idx[i, j, pl.ds(t * 16, 16)]
                src_idx[i, j, pl.ds(t * 16, 16)] = v * 2 + c

    def fire_gather(i):
        for j in range(KS):
            pltpu.async_copy(hs2x_hbm.at[src_idx.at[i, j]],
                             rows.at[i, j], gsem[i])

    def wait_gather(i):
        for j in range(KS):
            pltpu.make_async_copy(hs2x_hbm.at[src_idx.at[i, j]],
                                  rows.at[i, j], gsem[i]).wait()

    def fire_scatter(i):
        for j in range(KS):
            pltpu.async_copy(rows.at[i, j],
                             acc.at[dst_idx.at[i, j]], ssem[i], add=True)

    def wait_scatter(i):
        for j in range(KS):
            pltpu.make_async_copy(rows.at[i, j],
                                  acc.at[dst_idx.at[i, j]], ssem[i]).wait()

    load_idx(0, 0)
    fire_gather(0)
    load_idx(1, 1)
    fire_gather(1)

    def super_group(go, _):
        for i in range(SLOTS):
            g = SLOTS * go + i
            nxt = (i + 2) % SLOTS
            wait_gather(i)
            fire_scatter(i)
            if i < 2:
                @pl.when(go >= 1)
                def _():
                    wait_scatter(nxt)
            else:
                wait_scatter(nxt)

            @pl.when(g + 2 <= _G2 - 1)
            def _():
                load_idx(nxt, g + 2)
                fire_gather(nxt)
        return 0

    lax.fori_loop(0, _SUPER, super_group, 0, unroll=False)
    wait_scatter(2)
    wait_scatter(3)
    plsc.subcore_barrier()

    pltpu.sync_copy(acc.at[pl.ds(s * ROWS_PER_TILE, ROWS_PER_TILE)],
                    out_hbm.at[c, pl.ds(s * ROWS_PER_TILE, ROWS_PER_TILE)])


_TB = 1000


def _dinv_block(degp_ref):
    deg = degp_ref[0][:, 0:1] + degp_ref[1][:, 0:1] + 1.0
    return lax.rsqrt(deg)


def _scale_mm_body(degp_ref, x_ref, w_ref, out_ref):
    dinv = _dinv_block(degp_ref)
    h = jnp.dot(x_ref[...], w_ref[...], preferred_element_type=jnp.float32)
    out_ref[...] = h * dinv


def _mid_body(degp_ref, accp_ref, hs_ref, b_ref, w_ref, out_ref):
    dinv = _dinv_block(degp_ref)
    acc = jnp.concatenate([accp_ref[0], accp_ref[1]], axis=-1)
    h = jnp.maximum(dinv * (acc + hs_ref[...]) + b_ref[...], 0.0)
    out_ref[...] = jnp.dot(h, w_ref[...],
                           preferred_element_type=jnp.float32) * dinv


def _final_body(degp_ref, accp_ref, hs_ref, b_ref, mu_ref, lv_ref):
    dinv = _dinv_block(degp_ref)
    acc = jnp.concatenate([accp_ref[0], accp_ref[1]], axis=-1)
    out = dinv * (acc + hs_ref[...]) + b_ref[...]
    mu_ref[...] = out[:, :OUT_CH]
    lv_ref[...] = out[:, OUT_CH:]


_degp_spec = pl.BlockSpec((2, _TB, 16), lambda i: (0, i, 0))
_row_spec = pl.BlockSpec((_TB, HID_CH), lambda i: (i, 0))
_accp_spec = pl.BlockSpec((2, _TB, HALF), lambda i: (0, i, 0))
_w_spec = pl.BlockSpec((HID_CH, HID_CH), lambda i: (0, 0))
_b_spec = pl.BlockSpec((1, HID_CH), lambda i: (0, 0))
_half_spec = pl.BlockSpec((_TB, OUT_CH), lambda i: (i, 0))
_grid = (N // _TB,)
_row_out = jax.ShapeDtypeStruct((N, HID_CH), jnp.float32)

_scale_mm = pl.pallas_call(
    _scale_mm_body,
    grid=_grid,
    in_specs=[_degp_spec, _row_spec, _w_spec],
    out_specs=_row_spec,
    out_shape=_row_out,
)

_mid = pl.pallas_call(
    _mid_body,
    grid=_grid,
    in_specs=[_degp_spec, _accp_spec, _row_spec, _b_spec, _w_spec],
    out_specs=_row_spec,
    out_shape=_row_out,
)

_final = pl.pallas_call(
    _final_body,
    grid=_grid,
    in_specs=[_degp_spec, _accp_spec, _row_spec, _b_spec],
    out_specs=[_half_spec, _half_spec],
    out_shape=[jax.ShapeDtypeStruct((N, OUT_CH), jnp.float32),
               jax.ShapeDtypeStruct((N, OUT_CH), jnp.float32)],
)


def kernel(x, edge_index, W_shared, b_shared, W_mu, b_mu, W_logvar, b_logvar):
    npad_e = E_PAD - E
    pad_dst = N + (jnp.arange(npad_e, dtype=jnp.int32) % (N_PAD - N))
    pad_src = jnp.arange(npad_e, dtype=jnp.int32) % N
    srcm = jnp.concatenate([edge_index[0], pad_src]).reshape(EC_PAD, CHUNK)
    dstm = jnp.concatenate([edge_index[1], pad_dst]).reshape(EC_PAD, CHUNK)

    W_cat = jnp.concatenate([W_mu, W_logvar], axis=1)
    b_cat = jnp.concatenate([b_mu, b_logvar]).reshape(1, HID_CH)
    b_sh = b_shared.reshape(1, HID_CH)

    deg_kernel, agg_kernel = _sc_kernels()

    degp = deg_kernel(dstm)
    hs1 = _scale_mm(degp, x, W_shared)
    acc1 = agg_kernel(hs1.reshape(2 * N, HALF), srcm, dstm)
    hs2 = _mid(degp, acc1, hs1, b_sh, W_cat)
    acc2 = agg_kernel(hs2.reshape(2 * N, HALF), srcm, dstm)
    mu, logvar = _final(degp, acc2, hs2, b_cat)
    return (mu, logvar)
